# Initial kernel scaffold; baseline (speedup 1.0000x reference)
import jax
import jax.numpy as jnp
from jax.experimental import pallas as pl


def kernel(x, basis):
    raise NotImplementedError("write your pallas kernel here")


# speedup vs baseline: 8.9929x; 8.9929x over previous
import functools

import jax
import jax.numpy as jnp
import numpy as np
from jax.experimental import pallas as pl
from jax.experimental.pallas import tpu as pltpu

_NBITS = 2


def _enc_matrix():
    bitvecs = np.unpackbits(
        np.arange(2 ** _NBITS, dtype=np.uint8).reshape(-1, 1), axis=1
    )[:, -_NBITS:]
    return jnp.asarray(bitvecs.astype(np.float32) * 2.0 - 1.0)


def _bucketize_body(p_ref, x_ref, o_ref):
    v = x_ref[...]
    l0 = p_ref[0]
    l1 = p_ref[1]
    l2 = p_ref[2]
    l3 = p_ref[3]
    t0 = p_ref[4]
    t1 = p_ref[5]
    t2 = p_ref[6]
    lo = jnp.where(v > t0, l1, l0)
    hi = jnp.where(v > t2, l3, l2)
    o_ref[...] = jnp.where(v > t1, hi, lo)


def kernel(x, basis):
    qlevels = jnp.sort(_enc_matrix() @ basis)
    thres = (qlevels[:-1] + qlevels[1:]) * 0.5
    params = jnp.concatenate([qlevels, thres])

    rows, cols = 8192, 2048
    xf = x.reshape(rows, cols)
    block_rows = 512
    grid = (rows // block_rows,)

    out = pl.pallas_call(
        _bucketize_body,
        grid=grid,
        in_specs=[
            pl.BlockSpec(memory_space=pltpu.SMEM),
            pl.BlockSpec((block_rows, cols), lambda i: (i, 0)),
        ],
        out_specs=pl.BlockSpec((block_rows, cols), lambda i: (i, 0)),
        out_shape=jax.ShapeDtypeStruct((rows, cols), jnp.float32),
    )(params, xf)
    return out.reshape(x.shape)
